# baseline (speedup 1.0000x reference)
import functools

import jax
import jax.numpy as jnp
from jax import lax
from jax.experimental import pallas as pl
from jax.experimental.pallas import tpu as pltpu
from jax.experimental.pallas import tpu_sc as plsc

_TT = 16
_NBUF = 4

_NW = 32
_L = 16


def _sc_base_kernel(idx_hbm, exttab_hbm, slot_hbm, bias_hbm, out_hbm,
                    idx_v, rows_v, slot_v, bias_v, ebias_v, out_v, sem):
    rpw = out_v.shape[0]
    d = out_v.shape[1]
    n = slot_hbm.shape[0]
    nidx = idx_v.shape[0]
    wid = lax.axis_index("s") * 2 + lax.axis_index("c")
    r0 = wid * rpw
    n0 = lax.rem(r0, n)

    pltpu.sync_copy(idx_hbm.at[pl.ds(wid * nidx, nidx)], idx_v)
    gather = pltpu.make_async_copy(exttab_hbm.at[idx_v], rows_v, sem)
    gather.start()
    pltpu.sync_copy(slot_hbm.at[pl.ds(n0, rpw)], slot_v)
    pltpu.sync_copy(bias_hbm, bias_v)
    gather.wait()

    for g in range(d // _L):
        sl = pl.ds(g * _L, _L)
        ebias_v[sl] = rows_v[rpw, sl] + bias_v[sl]

    @pl.loop(0, rpw, unroll=4)
    def _row(j):
        for g in range(d // _L):
            sl = pl.ds(g * _L, _L)
            out_v[j, sl] = slot_v[j, sl] + rows_v[j, sl] + ebias_v[sl]

    pltpu.sync_copy(out_v, out_hbm.at[pl.ds(r0, rpw)])


def _stream_tile(i, nsteps, slot, dst, spikes_ref, wfull_ref, base_ref,
                 obuf_ref, sems):
    @pl.when(i >= _NBUF)
    def _wait_slot():
        pltpu.make_async_copy(obuf_ref.at[slot], dst, sems.at[slot]).wait()

    sp = spikes_ref[0]
    obuf_ref[slot] = (sp[:, None, :] * wfull_ref[...][None, :, :]
                      + base_ref[...][None, :, :])
    pltpu.make_async_copy(obuf_ref.at[slot], dst, sems.at[slot]).start()

    @pl.when(i == nsteps - 1)
    def _drain():
        for k in range(_NBUF):
            pltpu.make_async_copy(obuf_ref.at[k], dst, sems.at[k]).wait()


def _tc1_kernel(eids_ref, regions_ref, spikes_ref, wfull_ref, bcol_ref,
                slott_ref, regembt_ref, eidembt_ref, out_ref, base_ref,
                obuf_ref, sems):
    t_idx = pl.program_id(0)
    nsteps = pl.num_programs(0)
    slot = jax.lax.rem(t_idx, _NBUF)
    tt = obuf_ref.shape[1]
    d, n = base_ref.shape

    @pl.when(t_idx == 0)
    def _build_base():
        regions = regions_ref[0, :, :]
        nregions = regembt_ref.shape[1]
        oht = (jax.lax.broadcasted_iota(jnp.int32, (nregions, n), 0)
               == regions).astype(jnp.float32)
        regt = jnp.dot(regembt_ref[...], oht,
                       preferred_element_type=jnp.float32)
        e = eids_ref[0]
        neids = eidembt_ref.shape[1]
        ohe = (jax.lax.broadcasted_iota(jnp.int32, (neids, 8), 0) == e
               ).astype(jnp.float32)
        evt = jnp.dot(eidembt_ref[...], ohe,
                      preferred_element_type=jnp.float32)
        base_ref[...] = (slott_ref[...] + regt
                         + evt[:, 0:1] + bcol_ref[...])

    dst = out_ref.at[0, pl.ds(t_idx * tt, tt)]
    _stream_tile(t_idx, nsteps, slot, dst, spikes_ref, wfull_ref, base_ref,
                 obuf_ref, sems)


def _tc2_kernel(prev_ref, base_in_ref, spikes_ref, wfull_ref, out_ref,
                base_ref, obuf_ref, sems):
    b_idx = pl.program_id(0)
    t_idx = pl.program_id(1)
    nt = pl.num_programs(1)
    nsteps = pl.num_programs(0) * nt
    i = b_idx * nt + t_idx
    slot = jax.lax.rem(i, _NBUF)
    tt = obuf_ref.shape[1]

    @pl.when(t_idx == 0)
    def _load_base():
        base_ref[...] = base_in_ref[...].T

    dst = out_ref.at[b_idx + 1, pl.ds(t_idx * tt, tt)]
    _stream_tile(i, nsteps, slot, dst, spikes_ref, wfull_ref, base_ref,
                 obuf_ref, sems)


@jax.jit
def kernel(spikes, neuron_regions, eids, w_spike, b_spike, neuron_slot,
           region_emb, eid_emb):
    B, T, N = spikes.shape
    D = neuron_slot.shape[1]
    R = region_emb.shape[0]
    rows = B * N
    rpw = rows // _NW

    eids32 = eids.astype(jnp.int32)
    reg_part = neuron_regions.astype(jnp.int32).reshape(_NW, rpw)
    eid_part = jnp.broadcast_to(
        R + jnp.repeat(eids32, _NW // B)[:, None], (_NW, 8))
    idxflat = jnp.concatenate([reg_part, eid_part], axis=1).reshape(-1)
    exttab = jnp.pad(jnp.concatenate([region_emb, eid_emb], axis=0),
                     ((0, 0), (0, 128 - D)))

    scmesh = plsc.VectorSubcoreMesh(core_axis_name="c", subcore_axis_name="s")
    sc_base = functools.partial(
        pl.kernel,
        out_type=jax.ShapeDtypeStruct((rows, D), jnp.float32),
        mesh=scmesh,
        scratch_types=[
            pltpu.VMEM((rpw + 8,), jnp.int32),
            pltpu.VMEM((rpw + 8, 128), jnp.float32),
            pltpu.VMEM((rpw, D), jnp.float32),
            pltpu.VMEM((D,), jnp.float32),
            pltpu.VMEM((D,), jnp.float32),
            pltpu.VMEM((rpw, D), jnp.float32),
            pltpu.SemaphoreType.DMA,
        ],
    )(_sc_base_kernel)
    base = sc_base(idxflat, exttab, neuron_slot[:N], b_spike)

    wfull = jnp.broadcast_to(w_spike, (D, N))
    regions3 = neuron_regions.astype(jnp.int32).reshape(B, 1, N)
    bcol = b_spike.reshape(D, 1)
    slott = neuron_slot[:N].T
    regembt = region_emb.T
    eidembt = eid_emb.T

    common_scratch = [
        pltpu.VMEM((D, N), jnp.float32),
        pltpu.VMEM((_NBUF, _TT, D, N), jnp.float32),
        pltpu.SemaphoreType.DMA((_NBUF,)),
    ]

    out1 = pl.pallas_call(
        _tc1_kernel,
        grid=(T // _TT,),
        in_specs=[
            pl.BlockSpec(memory_space=pltpu.SMEM),
            pl.BlockSpec((1, 1, N), lambda t: (0, 0, 0)),
            pl.BlockSpec((1, _TT, N), lambda t: (0, t, 0)),
            pl.BlockSpec((D, N), lambda t: (0, 0)),
            pl.BlockSpec((D, 1), lambda t: (0, 0)),
            pl.BlockSpec((D, N), lambda t: (0, 0)),
            pl.BlockSpec((D, R), lambda t: (0, 0)),
            pl.BlockSpec((D, eid_emb.shape[0]), lambda t: (0, 0)),
        ],
        out_specs=pl.BlockSpec(memory_space=pltpu.MemorySpace.HBM),
        out_shape=jax.ShapeDtypeStruct((B, T, D, N), jnp.float32),
        scratch_shapes=common_scratch,
    )(eids32, regions3, spikes, wfull, bcol, slott, regembt, eidembt)

    outt = pl.pallas_call(
        _tc2_kernel,
        grid=(B - 1, T // _TT),
        in_specs=[
            pl.BlockSpec(memory_space=pltpu.MemorySpace.HBM),
            pl.BlockSpec((N, D), lambda b, t: (b + 1, 0)),
            pl.BlockSpec((1, _TT, N), lambda b, t: (b + 1, t, 0)),
            pl.BlockSpec((D, N), lambda b, t: (0, 0)),
        ],
        out_specs=pl.BlockSpec(memory_space=pltpu.MemorySpace.HBM),
        out_shape=jax.ShapeDtypeStruct((B, T, D, N), jnp.float32),
        scratch_shapes=common_scratch,
        input_output_aliases={0: 0},
    )(out1, base, spikes, wfull)
    return outt.transpose(0, 1, 3, 2)


# speedup vs baseline: 1.0828x; 1.0828x over previous
import functools

import jax
import jax.numpy as jnp
from jax import lax
from jax.experimental import pallas as pl
from jax.experimental.pallas import tpu as pltpu
from jax.experimental.pallas import tpu_sc as plsc

_TT = 16
_NBUF = 4

_NW = 32


def _sc_gather_kernel(idx_hbm, exttab_hbm, out_hbm, idx_v, rows_v, sem):
    rpw = rows_v.shape[0]
    wid = lax.axis_index("s") * 2 + lax.axis_index("c")
    r0 = wid * rpw

    pltpu.sync_copy(idx_hbm.at[pl.ds(r0, rpw)], idx_v)
    gather = pltpu.make_async_copy(exttab_hbm.at[idx_v], rows_v, sem)
    gather.start()
    gather.wait()
    pltpu.sync_copy(rows_v, out_hbm.at[pl.ds(r0, rpw)])


def _tc_kernel(eids_ref, regrows_ref, spikes_ref, wfull_ref, bcol_ref,
               slott_ref, eidembt_ref, out_ref, base_ref, obuf_ref, sems):
    b_idx = pl.program_id(0)
    t_idx = pl.program_id(1)
    nt = pl.num_programs(1)
    nsteps = pl.num_programs(0) * nt
    i = b_idx * nt + t_idx
    slot = jax.lax.rem(i, _NBUF)
    tt = obuf_ref.shape[1]
    d, n = base_ref.shape

    @pl.when(t_idx == 0)
    def _build_base():
        regt = regrows_ref[...][:, 0:d].T
        e = eids_ref[b_idx]
        neids = eidembt_ref.shape[1]
        ohe = (jax.lax.broadcasted_iota(jnp.int32, (neids, 8), 0) == e
               ).astype(jnp.float32)
        evt = jnp.dot(eidembt_ref[...], ohe,
                      preferred_element_type=jnp.float32)
        base_ref[...] = (slott_ref[...] + regt
                         + evt[:, 0:1] + bcol_ref[...])

    dst = out_ref.at[b_idx, pl.ds(t_idx * tt, tt)]

    @pl.when(i >= _NBUF)
    def _wait_slot():
        pltpu.make_async_copy(obuf_ref.at[slot], dst, sems.at[slot]).wait()

    sp = spikes_ref[0]
    obuf_ref[slot] = (sp[:, None, :] * wfull_ref[...][None, :, :]
                      + base_ref[...][None, :, :])
    pltpu.make_async_copy(obuf_ref.at[slot], dst, sems.at[slot]).start()

    @pl.when(i == nsteps - 1)
    def _drain():
        for k in range(_NBUF):
            pltpu.make_async_copy(obuf_ref.at[k], dst, sems.at[k]).wait()


@jax.jit
def kernel(spikes, neuron_regions, eids, w_spike, b_spike, neuron_slot,
           region_emb, eid_emb):
    B, T, N = spikes.shape
    D = neuron_slot.shape[1]
    rows = B * N
    rpw = rows // _NW

    eids32 = eids.astype(jnp.int32)
    idxflat = neuron_regions.astype(jnp.int32).reshape(-1)
    exttab = jnp.pad(region_emb, ((0, 0), (0, 128 - D)))

    scmesh = plsc.VectorSubcoreMesh(core_axis_name="c", subcore_axis_name="s")
    sc_gather = functools.partial(
        pl.kernel,
        out_type=jax.ShapeDtypeStruct((rows, 128), jnp.float32),
        mesh=scmesh,
        scratch_types=[
            pltpu.VMEM((rpw,), jnp.int32),
            pltpu.VMEM((rpw, 128), jnp.float32),
            pltpu.SemaphoreType.DMA,
        ],
    )(_sc_gather_kernel)
    regrows = sc_gather(idxflat, exttab)

    wfull = jnp.broadcast_to(w_spike, (D, N))
    bcol = b_spike.reshape(D, 1)
    slott = neuron_slot[:N].T
    eidembt = eid_emb.T

    outt = pl.pallas_call(
        _tc_kernel,
        grid=(B, T // _TT),
        in_specs=[
            pl.BlockSpec(memory_space=pltpu.SMEM),
            pl.BlockSpec((N, 128), lambda b, t: (b, 0)),
            pl.BlockSpec((1, _TT, N), lambda b, t: (b, t, 0)),
            pl.BlockSpec((D, N), lambda b, t: (0, 0)),
            pl.BlockSpec((D, 1), lambda b, t: (0, 0)),
            pl.BlockSpec((D, N), lambda b, t: (0, 0)),
            pl.BlockSpec((D, eid_emb.shape[0]), lambda b, t: (0, 0)),
        ],
        out_specs=pl.BlockSpec(memory_space=pltpu.MemorySpace.HBM),
        out_shape=jax.ShapeDtypeStruct((B, T, D, N), jnp.float32),
        scratch_shapes=[
            pltpu.VMEM((D, N), jnp.float32),
            pltpu.VMEM((_NBUF, _TT, D, N), jnp.float32),
            pltpu.SemaphoreType.DMA((_NBUF,)),
        ],
    )(eids32, regrows, spikes, wfull, bcol, slott, eidembt)
    return outt.transpose(0, 1, 3, 2)
